# baseline (speedup 1.0000x reference)
import functools

import jax
import jax.numpy as jnp
from jax import lax
from jax.experimental import pallas as pl
from jax.experimental.pallas import tpu as pltpu
from jax.experimental.pallas import tpu_sc as plsc

_VOCAB = 1000000
_DIM = 64
_BATCH = 4096
_SEQ = 200


def _positional_encoding(max_sequence_length, d_model):
    positions = jnp.arange(max_sequence_length)[:, None].astype(jnp.float32)
    dims = jnp.arange(d_model)[None, :]
    angle_rates = 1.0 / jnp.power(
        10000.0, (2 * (dims // 2)).astype(jnp.float32) / d_model
    )
    angle_rads = positions * angle_rates
    pe = jnp.zeros_like(angle_rads)
    pe = pe.at[:, 0::2].set(jnp.sin(angle_rads[:, 0::2]))
    pe = pe.at[:, 1::2].set(jnp.cos(angle_rads[:, 1::2]))
    return pe


def _make_sc_kernel(n_rows, seqs_per_worker):
    info = plsc.get_sparse_core_info()
    nc, ns = info.num_cores, info.num_subcores
    nw = nc * ns
    rows_per_worker = n_rows // nw
    mesh = plsc.VectorSubcoreMesh(core_axis_name="c", subcore_axis_name="s")

    @functools.partial(
        pl.kernel,
        mesh=mesh,
        compiler_params=pltpu.CompilerParams(use_tc_tiling_on_sc=False),
        out_type=jax.ShapeDtypeStruct((n_rows, _DIM), jnp.float32),
        scratch_types=[
            pltpu.VMEM((rows_per_worker,), jnp.int32),
            pltpu.VMEM((_SEQ, _DIM), jnp.float32),
            pltpu.VMEM((_SEQ, _DIM), jnp.float32),
            pltpu.SemaphoreType.DMA,
        ],
    )
    def sc_kernel(idx_hbm, pe_hbm, table_hbm, out_hbm, idx_v, pe_v, rows_v, sem):
        wid = lax.axis_index("s") * nc + lax.axis_index("c")
        base = wid * rows_per_worker
        pltpu.sync_copy(idx_hbm.at[pl.ds(base, rows_per_worker)], idx_v)
        pltpu.sync_copy(pe_hbm, pe_v)

        def chunk_body(i, carry):
            r0 = i * _SEQ
            cp1 = pltpu.async_copy(
                table_hbm.at[idx_v.at[pl.ds(r0, 128)]],
                rows_v.at[pl.ds(0, 128)],
                sem,
            )
            cp2 = pltpu.async_copy(
                table_hbm.at[idx_v.at[pl.ds(r0 + 128, _SEQ - 128)]],
                rows_v.at[pl.ds(128, _SEQ - 128)],
                sem,
            )
            cp1.wait()
            cp2.wait()

            def row_body(r, c2):
                for c in range(_DIM // 16):
                    sl = pl.ds(c * 16, 16)
                    rows_v[r, sl] = rows_v[r, sl] + pe_v[r, sl]
                return c2

            lax.fori_loop(0, _SEQ, row_body, 0, unroll=2)
            pltpu.sync_copy(rows_v, out_hbm.at[pl.ds(base + r0, _SEQ)])
            return carry

        lax.fori_loop(0, seqs_per_worker, chunk_body, 0)

    return sc_kernel


def kernel(input_ids, table):
    batch, seq = input_ids.shape
    n_rows = batch * seq
    idx_flat = input_ids.reshape(n_rows).astype(jnp.int32)
    pe = _positional_encoding(seq, table.shape[-1]).astype(jnp.float32)
    nw = 32
    seqs_per_worker = n_rows // (nw * seq)
    sc = _make_sc_kernel(n_rows, seqs_per_worker)
    out = sc(idx_flat, pe, table)
    return out.reshape(batch, seq, table.shape[-1])


# speedup vs baseline: 1.0149x; 1.0149x over previous
import functools

import jax
import jax.numpy as jnp
from jax import lax
from jax.experimental import pallas as pl
from jax.experimental.pallas import tpu as pltpu
from jax.experimental.pallas import tpu_sc as plsc

_VOCAB = 1000000
_DIM = 64
_BATCH = 4096
_SEQ = 200


def _positional_encoding(max_sequence_length, d_model):
    positions = jnp.arange(max_sequence_length)[:, None].astype(jnp.float32)
    dims = jnp.arange(d_model)[None, :]
    angle_rates = 1.0 / jnp.power(
        10000.0, (2 * (dims // 2)).astype(jnp.float32) / d_model
    )
    angle_rads = positions * angle_rates
    pe = jnp.zeros_like(angle_rads)
    pe = pe.at[:, 0::2].set(jnp.sin(angle_rads[:, 0::2]))
    pe = pe.at[:, 1::2].set(jnp.cos(angle_rads[:, 1::2]))
    return pe


def _make_sc_kernel(n_rows, seqs_per_worker):
    info = plsc.get_sparse_core_info()
    nc, ns = info.num_cores, info.num_subcores
    nw = nc * ns
    rows_per_worker = n_rows // nw
    mesh = plsc.VectorSubcoreMesh(core_axis_name="c", subcore_axis_name="s")

    @functools.partial(
        pl.kernel,
        mesh=mesh,
        compiler_params=pltpu.CompilerParams(use_tc_tiling_on_sc=False),
        out_type=jax.ShapeDtypeStruct((n_rows, _DIM), jnp.float32),
        scratch_types=[
            pltpu.VMEM((rows_per_worker,), jnp.int32),
            pltpu.VMEM((_SEQ, _DIM), jnp.float32),
            pltpu.VMEM((_SEQ, _DIM), jnp.float32),
            pltpu.SemaphoreType.DMA,
        ],
    )
    def sc_kernel(idx_hbm, pe_hbm, table_hbm, out_hbm, idx_v, pe_v, rows_v, sem):
        wid = lax.axis_index("s") * nc + lax.axis_index("c")
        base = wid * rows_per_worker
        pltpu.sync_copy(idx_hbm.at[pl.ds(base, rows_per_worker)], idx_v)
        pltpu.sync_copy(pe_hbm, pe_v)

        def chunk_body(i, carry):
            r0 = i * _SEQ
            pltpu.sync_copy(pe_hbm, rows_v)
            cp1 = pltpu.async_copy(
                table_hbm.at[idx_v.at[pl.ds(r0, 128)]],
                rows_v.at[pl.ds(0, 128)],
                sem,
                add=True,
            )
            cp2 = pltpu.async_copy(
                table_hbm.at[idx_v.at[pl.ds(r0 + 128, _SEQ - 128)]],
                rows_v.at[pl.ds(128, _SEQ - 128)],
                sem,
                add=True,
            )
            cp1.wait()
            cp2.wait()
            pltpu.sync_copy(rows_v, out_hbm.at[pl.ds(base + r0, _SEQ)])
            return carry

        lax.fori_loop(0, seqs_per_worker, chunk_body, 0)

    return sc_kernel


def kernel(input_ids, table):
    batch, seq = input_ids.shape
    n_rows = batch * seq
    idx_flat = input_ids.reshape(n_rows).astype(jnp.int32)
    pe = _positional_encoding(seq, table.shape[-1]).astype(jnp.float32)
    nw = 32
    seqs_per_worker = n_rows // (nw * seq)
    sc = _make_sc_kernel(n_rows, seqs_per_worker)
    out = sc(idx_flat, pe, table)
    return out.reshape(batch, seq, table.shape[-1])


# speedup vs baseline: 1.4504x; 1.4292x over previous
import functools

import jax
import jax.numpy as jnp
from jax import lax
from jax.experimental import pallas as pl
from jax.experimental.pallas import tpu as pltpu
from jax.experimental.pallas import tpu_sc as plsc

_VOCAB = 1000000
_DIM = 64
_BATCH = 4096
_SEQ = 200
_NSLOT = 4


def _positional_encoding(max_sequence_length, d_model):
    positions = jnp.arange(max_sequence_length)[:, None].astype(jnp.float32)
    dims = jnp.arange(d_model)[None, :]
    angle_rates = 1.0 / jnp.power(
        10000.0, (2 * (dims // 2)).astype(jnp.float32) / d_model
    )
    angle_rads = positions * angle_rates
    pe = jnp.zeros_like(angle_rads)
    pe = pe.at[:, 0::2].set(jnp.sin(angle_rads[:, 0::2]))
    pe = pe.at[:, 1::2].set(jnp.cos(angle_rads[:, 1::2]))
    return pe


def _make_sc_kernel(n_rows, n_chunks):
    info = plsc.get_sparse_core_info()
    nc, ns = info.num_cores, info.num_subcores
    nw = nc * ns
    rows_per_worker = n_rows // nw
    mesh = plsc.VectorSubcoreMesh(core_axis_name="c", subcore_axis_name="s")

    @functools.partial(
        pl.kernel,
        mesh=mesh,
        compiler_params=pltpu.CompilerParams(use_tc_tiling_on_sc=False),
        out_type=jax.ShapeDtypeStruct((n_rows, _DIM), jnp.float32),
        scratch_types=[
            pltpu.VMEM((rows_per_worker,), jnp.int32),
            pltpu.VMEM((_NSLOT * _SEQ, _DIM), jnp.float32),
            pltpu.VMEM_SHARED((_SEQ, _DIM), jnp.float32),
            pltpu.SemaphoreType.DMA((_NSLOT,)),
            pltpu.SemaphoreType.DMA((_NSLOT,)),
            pltpu.SemaphoreType.DMA((_NSLOT,)),
            pltpu.SemaphoreType.DMA,
        ],
    )
    def sc_kernel(
        idx_hbm, pe_hbm, table_hbm, out_hbm,
        idx_v, rows_v, pe_sh, fsem, gsem, wsem, sem0,
    ):
        wid = lax.axis_index("s") * nc + lax.axis_index("c")
        base = wid * rows_per_worker

        @pl.when(lax.axis_index("s") == 0)
        def _():
            pltpu.async_copy(pe_hbm, pe_sh, sem0).wait()

        pltpu.sync_copy(idx_hbm.at[pl.ds(base, rows_per_worker)], idx_v)
        plsc.subcore_barrier()

        def slot_ref(b):
            return rows_v.at[pl.ds(b * _SEQ, _SEQ)]

        def fire_fill(b):
            pltpu.make_async_copy(pe_sh, slot_ref(b), fsem.at[b]).start()

        def wait_fill(b):
            pltpu.make_async_copy(pe_sh, slot_ref(b), fsem.at[b]).wait()

        def fire_gather(i, b):
            r0 = i * _SEQ
            pltpu.make_async_copy(
                table_hbm.at[idx_v.at[pl.ds(r0, 128)]],
                rows_v.at[pl.ds(b * _SEQ, 128)],
                gsem.at[b],
            ).start(add=True)
            pltpu.make_async_copy(
                table_hbm.at[idx_v.at[pl.ds(r0 + 128, _SEQ - 128)]],
                rows_v.at[pl.ds(b * _SEQ + 128, _SEQ - 128)],
                gsem.at[b],
            ).start(add=True)

        def wait_gather(b):
            pltpu.make_async_copy(
                table_hbm.at[pl.ds(0, 128)],
                rows_v.at[pl.ds(b * _SEQ, 128)],
                gsem.at[b],
            ).wait()
            pltpu.make_async_copy(
                table_hbm.at[pl.ds(0, _SEQ - 128)],
                rows_v.at[pl.ds(b * _SEQ + 128, _SEQ - 128)],
                gsem.at[b],
            ).wait()

        def fire_write(i, b):
            pltpu.make_async_copy(
                slot_ref(b), out_hbm.at[pl.ds(base + i * _SEQ, _SEQ)], wsem.at[b]
            ).start()

        def wait_write(b):
            pltpu.make_async_copy(
                slot_ref(b), out_hbm.at[pl.ds(0, _SEQ)], wsem.at[b]
            ).wait()

        fire_fill(0)
        fire_fill(1)

        def group_body(j, carry):
            for b in range(_NSLOT):
                i = j * _NSLOT + b
                wait_fill(b)
                fire_gather(i, b)
                bp = (b - 1) % _NSLOT

                @pl.when(i > 0)
                def _():
                    wait_gather(bp)
                    fire_write(i - 1, bp)

                bn = (b + 2) % _NSLOT

                @pl.when(i + 2 < n_chunks)
                def _():
                    @pl.when(i >= 2)
                    def _():
                        wait_write(bn)

                    fire_fill_i = pltpu.make_async_copy(
                        pe_sh, slot_ref(bn), fsem.at[bn]
                    )
                    fire_fill_i.start()

            return carry

        lax.fori_loop(0, n_chunks // _NSLOT, group_body, 0)

        blast = (n_chunks - 1) % _NSLOT
        wait_gather(blast)
        fire_write(n_chunks - 1, blast)
        for b in range(_NSLOT):
            wait_write(b)

    return sc_kernel


def kernel(input_ids, table):
    batch, seq = input_ids.shape
    n_rows = batch * seq
    idx_flat = input_ids.reshape(n_rows).astype(jnp.int32)
    pe = _positional_encoding(seq, table.shape[-1]).astype(jnp.float32)
    nw = 32
    n_chunks = n_rows // (nw * seq)
    sc = _make_sc_kernel(n_rows, n_chunks)
    out = sc(idx_flat, pe, table)
    return out.reshape(batch, seq, table.shape[-1])


# speedup vs baseline: 1.4517x; 1.0009x over previous
import functools

import jax
import jax.numpy as jnp
from jax import lax
from jax.experimental import pallas as pl
from jax.experimental.pallas import tpu as pltpu
from jax.experimental.pallas import tpu_sc as plsc

_DIM = 64
_SEQ = 200
_NSLOT = 4


def _positional_encoding(max_sequence_length, d_model):
    positions = jnp.arange(max_sequence_length)[:, None].astype(jnp.float32)
    dims = jnp.arange(d_model)[None, :]
    angle_rates = 1.0 / jnp.power(
        10000.0, (2 * (dims // 2)).astype(jnp.float32) / d_model
    )
    angle_rads = positions * angle_rates
    pe = jnp.zeros_like(angle_rads)
    pe = pe.at[:, 0::2].set(jnp.sin(angle_rads[:, 0::2]))
    pe = pe.at[:, 1::2].set(jnp.cos(angle_rads[:, 1::2]))
    return pe


def _make_sc_kernel(batch, n_chunks):
    info = plsc.get_sparse_core_info()
    nc, ns = info.num_cores, info.num_subcores
    mesh = plsc.VectorSubcoreMesh(core_axis_name="c", subcore_axis_name="s")

    @functools.partial(
        pl.kernel,
        mesh=mesh,
        compiler_params=pltpu.CompilerParams(use_tc_tiling_on_sc=False),
        out_type=jax.ShapeDtypeStruct((batch, _SEQ, _DIM), jnp.float32),
        scratch_types=[
            pltpu.VMEM((n_chunks, _SEQ), jnp.int32),
            pltpu.VMEM((_NSLOT * _SEQ, _DIM), jnp.float32),
            pltpu.VMEM_SHARED((_SEQ, _DIM), jnp.float32),
            pltpu.SemaphoreType.DMA((_NSLOT,)),
            pltpu.SemaphoreType.DMA((_NSLOT,)),
            pltpu.SemaphoreType.DMA((_NSLOT,)),
            pltpu.SemaphoreType.DMA,
        ],
    )
    def sc_kernel(
        idx_hbm, pe_hbm, table_hbm, out_hbm,
        idx_v, rows_v, pe_sh, fsem, gsem, wsem, sem0,
    ):
        wid = lax.axis_index("s") * nc + lax.axis_index("c")
        base = wid * n_chunks

        @pl.when(lax.axis_index("s") == 0)
        def _():
            pltpu.async_copy(pe_hbm, pe_sh, sem0).wait()

        pltpu.sync_copy(idx_hbm.at[pl.ds(base, n_chunks)], idx_v)
        plsc.subcore_barrier()

        def slot_ref(b):
            return rows_v.at[pl.ds(b * _SEQ, _SEQ)]

        def fire_fill(b):
            pltpu.make_async_copy(pe_sh, slot_ref(b), fsem.at[b]).start()

        def wait_fill(b):
            pltpu.make_async_copy(pe_sh, slot_ref(b), fsem.at[b]).wait()

        def fire_gather(i, b):
            pltpu.make_async_copy(
                table_hbm.at[idx_v.at[i, pl.ds(0, 128)]],
                rows_v.at[pl.ds(b * _SEQ, 128)],
                gsem.at[b],
            ).start(add=True)
            pltpu.make_async_copy(
                table_hbm.at[idx_v.at[i, pl.ds(128, _SEQ - 128)]],
                rows_v.at[pl.ds(b * _SEQ + 128, _SEQ - 128)],
                gsem.at[b],
            ).start(add=True)

        def wait_gather(b):
            pltpu.make_async_copy(
                table_hbm.at[pl.ds(0, 128)],
                rows_v.at[pl.ds(b * _SEQ, 128)],
                gsem.at[b],
            ).wait()
            pltpu.make_async_copy(
                table_hbm.at[pl.ds(0, _SEQ - 128)],
                rows_v.at[pl.ds(b * _SEQ + 128, _SEQ - 128)],
                gsem.at[b],
            ).wait()

        def fire_write(i, b):
            pltpu.make_async_copy(slot_ref(b), out_hbm.at[base + i], wsem.at[b]).start()

        def wait_write(b):
            pltpu.make_async_copy(slot_ref(b), out_hbm.at[0], wsem.at[b]).wait()

        fire_fill(0)
        fire_fill(1)

        def group_body(j, carry):
            for b in range(_NSLOT):
                i = j * _NSLOT + b
                wait_fill(b)
                fire_gather(i, b)
                bp = (b - 1) % _NSLOT

                @pl.when(i > 0)
                def _():
                    wait_gather(bp)
                    fire_write(i - 1, bp)

                bn = (b + 2) % _NSLOT

                @pl.when(i + 2 < n_chunks)
                def _():
                    @pl.when(i >= 2)
                    def _():
                        wait_write(bn)

                    fire_fill(bn)

            return carry

        lax.fori_loop(0, n_chunks // _NSLOT, group_body, 0)

        blast = (n_chunks - 1) % _NSLOT
        wait_gather(blast)
        fire_write(n_chunks - 1, blast)
        for b in range(_NSLOT):
            wait_write(b)

    return sc_kernel


def kernel(input_ids, table):
    batch, seq = input_ids.shape
    ids = input_ids if input_ids.dtype == jnp.int32 else input_ids.astype(jnp.int32)
    pe = _positional_encoding(seq, table.shape[-1]).astype(jnp.float32)
    n_chunks = batch // 32
    sc = _make_sc_kernel(batch, n_chunks)
    return sc(ids, pe, table)


# speedup vs baseline: 1.7662x; 1.2167x over previous
import functools

import jax
import jax.numpy as jnp
from jax import lax
from jax.experimental import pallas as pl
from jax.experimental.pallas import tpu as pltpu
from jax.experimental.pallas import tpu_sc as plsc

_DIM = 64
_WIDE = 128
_SEQ = 200
_NSLOT = 4
_GRP = 4


def _positional_encoding(max_sequence_length, d_model):
    positions = jnp.arange(max_sequence_length)[:, None].astype(jnp.float32)
    dims = jnp.arange(d_model)[None, :]
    angle_rates = 1.0 / jnp.power(
        10000.0, (2 * (dims // 2)).astype(jnp.float32) / d_model
    )
    angle_rads = positions * angle_rates
    pe = jnp.zeros_like(angle_rads)
    pe = pe.at[:, 0::2].set(jnp.sin(angle_rads[:, 0::2]))
    pe = pe.at[:, 1::2].set(jnp.cos(angle_rads[:, 1::2]))
    return pe


def _make_sc_kernel(batch, n_chunks):
    info = plsc.get_sparse_core_info()
    nc, ns = info.num_cores, info.num_subcores
    mesh = plsc.VectorSubcoreMesh(core_axis_name="c", subcore_axis_name="s")
    n_groups = n_chunks // _GRP

    @functools.partial(
        pl.kernel,
        mesh=mesh,
        compiler_params=pltpu.CompilerParams(use_tc_tiling_on_sc=False),
        out_type=jax.ShapeDtypeStruct((batch, _SEQ, _WIDE), jnp.float32),
        scratch_types=[
            pltpu.VMEM((2, _GRP, _SEQ), jnp.int32),
            pltpu.VMEM((_NSLOT * _SEQ, _WIDE), jnp.float32),
            pltpu.VMEM_SHARED((_SEQ, _WIDE), jnp.float32),
            pltpu.SemaphoreType.DMA((_NSLOT,)),
            pltpu.SemaphoreType.DMA((_NSLOT,)),
            pltpu.SemaphoreType.DMA((_NSLOT,)),
            pltpu.SemaphoreType.DMA((2,)),
            pltpu.SemaphoreType.DMA,
        ],
    )
    def sc_kernel(
        idx_hbm, pe_hbm, table_hbm, out_hbm,
        idx_v, rows_v, pe_sh, fsem, gsem, wsem, isem, sem0,
    ):
        wid = lax.axis_index("s") * nc + lax.axis_index("c")
        base = wid * n_chunks

        @pl.when(lax.axis_index("s") == 0)
        def _():
            pltpu.async_copy(pe_hbm, pe_sh, sem0).wait()

        plsc.subcore_barrier()

        def fire_idx(j, k):
            pltpu.make_async_copy(
                idx_hbm.at[pl.ds(base + j * _GRP, _GRP)], idx_v.at[k], isem.at[k]
            ).start()

        def wait_idx(k):
            pltpu.make_async_copy(
                idx_hbm.at[pl.ds(0, _GRP)], idx_v.at[k], isem.at[k]
            ).wait()

        def slot_ref(b):
            return rows_v.at[pl.ds(b * _SEQ, _SEQ)]

        def fire_fill(b):
            pltpu.make_async_copy(pe_sh, slot_ref(b), fsem.at[b]).start()

        def wait_fill(b):
            pltpu.make_async_copy(pe_sh, slot_ref(b), fsem.at[b]).wait()

        def fire_gather(k, b):
            pltpu.make_async_copy(
                table_hbm.at[idx_v.at[k, b, pl.ds(0, 128)]],
                rows_v.at[pl.ds(b * _SEQ, 128)],
                gsem.at[b],
            ).start(add=True)
            pltpu.make_async_copy(
                table_hbm.at[idx_v.at[k, b, pl.ds(128, _SEQ - 128)]],
                rows_v.at[pl.ds(b * _SEQ + 128, _SEQ - 128)],
                gsem.at[b],
            ).start(add=True)

        def wait_gather(b):
            pltpu.make_async_copy(
                table_hbm.at[pl.ds(0, 128)],
                rows_v.at[pl.ds(b * _SEQ, 128)],
                gsem.at[b],
            ).wait()
            pltpu.make_async_copy(
                table_hbm.at[pl.ds(0, _SEQ - 128)],
                rows_v.at[pl.ds(b * _SEQ + 128, _SEQ - 128)],
                gsem.at[b],
            ).wait()

        def fire_write(i, b):
            pltpu.make_async_copy(slot_ref(b), out_hbm.at[base + i], wsem.at[b]).start()

        def wait_write(b):
            pltpu.make_async_copy(slot_ref(b), out_hbm.at[0], wsem.at[b]).wait()

        fire_idx(0, 0)
        fire_idx(1, 1)
        fire_fill(0)
        fire_fill(1)

        def group_body(j, carry):
            k = lax.rem(j, 2)
            wait_idx(k)
            for b in range(_GRP):
                i = j * _GRP + b
                wait_fill(b)
                fire_gather(k, b)
                bp = (b - 1) % _NSLOT

                @pl.when(i > 0)
                def _():
                    wait_gather(bp)
                    fire_write(i - 1, bp)

                if b == 0:
                    @pl.when((j >= 1) & (j + 1 < n_groups))
                    def _():
                        fire_idx(j + 1, 1 - k)

                bn = (b + 2) % _NSLOT

                @pl.when(i + 2 < n_chunks)
                def _():
                    @pl.when(i >= 2)
                    def _():
                        wait_write(bn)

                    fire_fill(bn)

            return carry

        lax.fori_loop(0, n_groups, group_body, 0)

        blast = (n_chunks - 1) % _NSLOT
        wait_gather(blast)
        fire_write(n_chunks - 1, blast)
        for b in range(_NSLOT):
            wait_write(b)

    return sc_kernel


def kernel(input_ids, table):
    batch, seq = input_ids.shape
    ids = input_ids if input_ids.dtype == jnp.int32 else input_ids.astype(jnp.int32)
    pe = _positional_encoding(seq, table.shape[-1]).astype(jnp.float32)
    pe = jnp.pad(pe, ((0, 0), (0, _WIDE - _DIM)))
    table = jnp.pad(table, ((0, 0), (0, _WIDE - _DIM)))
    n_chunks = batch // 32
    sc = _make_sc_kernel(batch, n_chunks)
    out = sc(ids, pe, table)
    return lax.slice(out, (0, 0, 0), (batch, seq, _DIM))


# speedup vs baseline: 1.9279x; 1.0915x over previous
import functools

import jax
import jax.numpy as jnp
from jax import lax
from jax.experimental import pallas as pl
from jax.experimental.pallas import tpu as pltpu
from jax.experimental.pallas import tpu_sc as plsc

_DIM = 64
_WIDE = 128
_SEQ = 200
_NSLOT = 4
_GRP = 4


def _positional_encoding(max_sequence_length, d_model):
    positions = jnp.arange(max_sequence_length)[:, None].astype(jnp.float32)
    dims = jnp.arange(d_model)[None, :]
    angle_rates = 1.0 / jnp.power(
        10000.0, (2 * (dims // 2)).astype(jnp.float32) / d_model
    )
    angle_rads = positions * angle_rates
    pe = jnp.zeros_like(angle_rads)
    pe = pe.at[:, 0::2].set(jnp.sin(angle_rads[:, 0::2]))
    pe = pe.at[:, 1::2].set(jnp.cos(angle_rads[:, 1::2]))
    return pe


def _make_sc_kernel(batch, n_chunks):
    info = plsc.get_sparse_core_info()
    nc, ns = info.num_cores, info.num_subcores
    mesh = plsc.VectorSubcoreMesh(core_axis_name="c", subcore_axis_name="s")
    n_groups = n_chunks // _GRP

    @functools.partial(
        pl.kernel,
        mesh=mesh,
        compiler_params=pltpu.CompilerParams(use_tc_tiling_on_sc=False),
        out_type=jax.ShapeDtypeStruct((batch, _SEQ, _WIDE), jnp.float32),
        scratch_types=[
            pltpu.VMEM((2, _GRP, _SEQ), jnp.int32),
            pltpu.VMEM((_NSLOT * _SEQ, _DIM), jnp.float32),
            pltpu.VMEM_SHARED((_SEQ, _DIM), jnp.float32),
            pltpu.SemaphoreType.DMA((_NSLOT,)),
            pltpu.SemaphoreType.DMA((_NSLOT,)),
            pltpu.SemaphoreType.DMA((_NSLOT,)),
            pltpu.SemaphoreType.DMA((2,)),
            pltpu.SemaphoreType.DMA,
        ],
    )
    def sc_kernel(
        idx_hbm, pe_hbm, table_hbm, out_hbm,
        idx_v, rows_v, pe_sh, fsem, gsem, wsem, isem, sem0,
    ):
        wid = lax.axis_index("s") * nc + lax.axis_index("c")
        base = wid * n_chunks

        @pl.when(lax.axis_index("s") == 0)
        def _():
            pltpu.async_copy(pe_hbm, pe_sh, sem0).wait()

        plsc.subcore_barrier()

        def fire_idx(j, k):
            pltpu.make_async_copy(
                idx_hbm.at[pl.ds(base + j * _GRP, _GRP)], idx_v.at[k], isem.at[k]
            ).start()

        def wait_idx(k):
            pltpu.make_async_copy(
                idx_hbm.at[pl.ds(0, _GRP)], idx_v.at[k], isem.at[k]
            ).wait()

        def slot_ref(b):
            return rows_v.at[pl.ds(b * _SEQ, _SEQ)]

        def fire_fill(b):
            pltpu.make_async_copy(pe_sh, slot_ref(b), fsem.at[b]).start()

        def wait_fill(b):
            pltpu.make_async_copy(pe_sh, slot_ref(b), fsem.at[b]).wait()

        def fire_gather(k, b):
            pltpu.make_async_copy(
                table_hbm.at[idx_v.at[k, b, pl.ds(0, 128)]],
                rows_v.at[pl.ds(b * _SEQ, 128)],
                gsem.at[b],
            ).start(add=True)
            pltpu.make_async_copy(
                table_hbm.at[idx_v.at[k, b, pl.ds(128, _SEQ - 128)]],
                rows_v.at[pl.ds(b * _SEQ + 128, _SEQ - 128)],
                gsem.at[b],
            ).start(add=True)

        def wait_gather(b):
            pltpu.make_async_copy(
                table_hbm.at[pl.ds(0, 128)],
                rows_v.at[pl.ds(b * _SEQ, 128)],
                gsem.at[b],
            ).wait()
            pltpu.make_async_copy(
                table_hbm.at[pl.ds(0, _SEQ - 128)],
                rows_v.at[pl.ds(b * _SEQ + 128, _SEQ - 128)],
                gsem.at[b],
            ).wait()

        def fire_write(i, b):
            pltpu.make_async_copy(
                slot_ref(b),
                out_hbm.at[base + i, pl.ds(0, _SEQ), pl.ds(0, _DIM)],
                wsem.at[b],
            ).start()

        def wait_write(b):
            pltpu.make_async_copy(
                slot_ref(b),
                out_hbm.at[0, pl.ds(0, _SEQ), pl.ds(0, _DIM)],
                wsem.at[b],
            ).wait()

        fire_idx(0, 0)
        fire_idx(1, 1)
        fire_fill(0)
        fire_fill(1)

        def group_body(j, carry):
            k = lax.rem(j, 2)
            wait_idx(k)
            for b in range(_GRP):
                i = j * _GRP + b
                wait_fill(b)
                fire_gather(k, b)
                bp = (b - 1) % _NSLOT

                @pl.when(i > 0)
                def _():
                    wait_gather(bp)
                    fire_write(i - 1, bp)

                if b == 0:
                    @pl.when((j >= 1) & (j + 1 < n_groups))
                    def _():
                        fire_idx(j + 1, 1 - k)

                bn = (b + 2) % _NSLOT

                @pl.when(i + 2 < n_chunks)
                def _():
                    @pl.when(i >= 2)
                    def _():
                        wait_write(bn)

                    fire_fill(bn)

            return carry

        lax.fori_loop(0, n_groups, group_body, 0)

        blast = (n_chunks - 1) % _NSLOT
        wait_gather(blast)
        fire_write(n_chunks - 1, blast)
        for b in range(_NSLOT):
            wait_write(b)

    return sc_kernel


def kernel(input_ids, table):
    batch, seq = input_ids.shape
    ids = input_ids if input_ids.dtype == jnp.int32 else input_ids.astype(jnp.int32)
    pe = _positional_encoding(seq, table.shape[-1]).astype(jnp.float32)
    n_chunks = batch // 32
    sc = _make_sc_kernel(batch, n_chunks)
    out = sc(ids, pe, table)
    return lax.slice(out, (0, 0, 0), (batch, seq, _DIM))


# speedup vs baseline: 2.2143x; 1.1485x over previous
import functools

import jax
import jax.numpy as jnp
from jax import lax
from jax.experimental import pallas as pl
from jax.experimental.pallas import tpu as pltpu
from jax.experimental.pallas import tpu_sc as plsc

_DIM = 64
_WIDE = 128
_SEQ = 200
_NSLOT = 4
_GRP = 4


def _positional_encoding(max_sequence_length, d_model):
    positions = jnp.arange(max_sequence_length)[:, None].astype(jnp.float32)
    dims = jnp.arange(d_model)[None, :]
    angle_rates = 1.0 / jnp.power(
        10000.0, (2 * (dims // 2)).astype(jnp.float32) / d_model
    )
    angle_rads = positions * angle_rates
    pe = jnp.zeros_like(angle_rads)
    pe = pe.at[:, 0::2].set(jnp.sin(angle_rads[:, 0::2]))
    pe = pe.at[:, 1::2].set(jnp.cos(angle_rads[:, 1::2]))
    return pe


def _make_sc_kernel(batch, n_chunks):
    info = plsc.get_sparse_core_info()
    nc, ns = info.num_cores, info.num_subcores
    mesh = plsc.VectorSubcoreMesh(core_axis_name="c", subcore_axis_name="s")
    n_groups = n_chunks // _GRP

    @functools.partial(
        pl.kernel,
        mesh=mesh,
        compiler_params=pltpu.CompilerParams(use_tc_tiling_on_sc=False),
        out_type=jax.ShapeDtypeStruct((batch, _SEQ, _WIDE), jnp.float32),
        scratch_types=[
            pltpu.VMEM((2, _GRP, _SEQ), jnp.int32),
            pltpu.VMEM((_NSLOT * _SEQ, _DIM), jnp.float32),
            pltpu.VMEM_SHARED((_SEQ, _DIM), jnp.float32),
            pltpu.SemaphoreType.DMA((_NSLOT,)),
            pltpu.SemaphoreType.DMA((_NSLOT,)),
            pltpu.SemaphoreType.DMA((_NSLOT,)),
            pltpu.SemaphoreType.DMA((2,)),
            pltpu.SemaphoreType.DMA,
        ],
    )
    def sc_kernel(
        idx_hbm, pe_hbm, table_hbm, out_hbm,
        idx_v, rows_v, pe_sh, fsem, gsem, wsem, isem, sem0,
    ):
        wid = lax.axis_index("s") * nc + lax.axis_index("c")
        base = wid * n_chunks

        @pl.when(lax.axis_index("s") == 0)
        def _():
            pltpu.async_copy(pe_hbm, pe_sh, sem0).wait()

        plsc.subcore_barrier()

        def fire_idx(j, k):
            pltpu.make_async_copy(
                idx_hbm.at[pl.ds(base + j * _GRP, _GRP)], idx_v.at[k], isem.at[k]
            ).start()

        def wait_idx(k):
            pltpu.make_async_copy(
                idx_hbm.at[pl.ds(0, _GRP)], idx_v.at[k], isem.at[k]
            ).wait()

        def slot_ref(b):
            return rows_v.at[pl.ds(b * _SEQ, _SEQ)]

        def fire_fill(b):
            pltpu.make_async_copy(pe_sh, slot_ref(b), fsem.at[b]).start()

        def wait_fill(b):
            pltpu.make_async_copy(pe_sh, slot_ref(b), fsem.at[b]).wait()

        def fire_gather(k, b):
            pltpu.make_async_copy(
                table_hbm.at[idx_v.at[k, b, pl.ds(0, 128)]],
                rows_v.at[pl.ds(b * _SEQ, 128)],
                gsem.at[b],
            ).start(add=True)
            pltpu.make_async_copy(
                table_hbm.at[idx_v.at[k, b, pl.ds(128, _SEQ - 128)]],
                rows_v.at[pl.ds(b * _SEQ + 128, _SEQ - 128)],
                gsem.at[b],
            ).start(add=True)

        def wait_gather(b):
            pltpu.make_async_copy(
                table_hbm.at[pl.ds(0, 128)],
                rows_v.at[pl.ds(b * _SEQ, 128)],
                gsem.at[b],
            ).wait()
            pltpu.make_async_copy(
                table_hbm.at[pl.ds(0, _SEQ - 128)],
                rows_v.at[pl.ds(b * _SEQ + 128, _SEQ - 128)],
                gsem.at[b],
            ).wait()

        def fire_write(i, b):
            pltpu.make_async_copy(
                slot_ref(b),
                out_hbm.at[base + i, pl.ds(0, _SEQ), pl.ds(0, _DIM)],
                wsem.at[b],
            ).start()

        def wait_write(b):
            pltpu.make_async_copy(
                slot_ref(b),
                out_hbm.at[0, pl.ds(0, _SEQ), pl.ds(0, _DIM)],
                wsem.at[b],
            ).wait()

        fire_idx(0, 0)
        fire_idx(1, 1)
        fire_fill(0)
        fire_fill(1)

        def group_body(j, carry):
            k = lax.rem(j, 2)
            wait_idx(k)
            for b in range(_GRP):
                i = j * _GRP + b
                wait_fill(b)
                fire_gather(k, b)
                bp = (b - 1) % _NSLOT

                @pl.when(i > 0)
                def _():
                    wait_gather(bp)
                    fire_write(i - 1, bp)

                if b == 0:
                    @pl.when((j >= 1) & (j + 1 < n_groups))
                    def _():
                        fire_idx(j + 1, 1 - k)

                bn = (b + 2) % _NSLOT

                @pl.when(i + 2 < n_chunks)
                def _():
                    @pl.when(i >= 2)
                    def _():
                        wait_write(bn)

                    fire_fill(bn)

            return carry

        lax.fori_loop(0, n_groups, group_body, 0)

        blast = (n_chunks - 1) % _NSLOT
        wait_gather(blast)
        fire_write(n_chunks - 1, blast)
        for b in range(_NSLOT):
            wait_write(b)

    return sc_kernel


def _transpose_pad_table(table):
    v = table.shape[0]
    tt = table.T
    blk = 2048
    grid = (v + blk - 1) // blk

    def body(in_ref, out_ref):
        out_ref[:, 0:_DIM] = in_ref[...].T

    return pl.pallas_call(
        body,
        grid=(grid,),
        in_specs=[pl.BlockSpec((_DIM, blk), lambda i: (0, i))],
        out_specs=pl.BlockSpec((blk, _WIDE), lambda i: (i, 0)),
        out_shape=jax.ShapeDtypeStruct((v, _WIDE), jnp.float32),
    )(tt)


def kernel(input_ids, table):
    batch, seq = input_ids.shape
    ids = input_ids if input_ids.dtype == jnp.int32 else input_ids.astype(jnp.int32)
    pe = _positional_encoding(seq, table.shape[-1]).astype(jnp.float32)
    n_chunks = batch // 32
    tpad = _transpose_pad_table(table)
    trows = jnp.reshape(tpad, (2 * table.shape[0], _DIM))
    ids2 = ids * 2
    sc = _make_sc_kernel(batch, n_chunks)
    out = sc(ids2, pe, trows)
    return lax.slice(out, (0, 0, 0), (batch, seq, _DIM))


# speedup vs baseline: 2.9294x; 1.3230x over previous
import functools

import jax
import jax.numpy as jnp
from jax import lax
from jax.experimental import pallas as pl
from jax.experimental.pallas import tpu as pltpu
from jax.experimental.pallas import tpu_sc as plsc

_DIM = 64
_WIDE = 128
_SEQ = 200
_NSLOT = 4
_GRP = 4


def _positional_encoding(max_sequence_length, d_model):
    positions = jnp.arange(max_sequence_length)[:, None].astype(jnp.float32)
    dims = jnp.arange(d_model)[None, :]
    angle_rates = 1.0 / jnp.power(
        10000.0, (2 * (dims // 2)).astype(jnp.float32) / d_model
    )
    angle_rads = positions * angle_rates
    pe = jnp.zeros_like(angle_rads)
    pe = pe.at[:, 0::2].set(jnp.sin(angle_rads[:, 0::2]))
    pe = pe.at[:, 1::2].set(jnp.cos(angle_rads[:, 1::2]))
    return pe


def _make_sc_kernel(batch, n_chunks):
    info = plsc.get_sparse_core_info()
    nc, ns = info.num_cores, info.num_subcores
    mesh = plsc.VectorSubcoreMesh(core_axis_name="c", subcore_axis_name="s")
    n_groups = n_chunks // _GRP

    @functools.partial(
        pl.kernel,
        mesh=mesh,
        compiler_params=pltpu.CompilerParams(use_tc_tiling_on_sc=False),
        out_type=jax.ShapeDtypeStruct((batch, _SEQ, _WIDE), jnp.float32),
        scratch_types=[
            pltpu.VMEM((2, _GRP, _SEQ), jnp.int32),
            pltpu.VMEM((_NSLOT * _SEQ, _DIM), jnp.float32),
            pltpu.VMEM_SHARED((_SEQ, _DIM), jnp.float32),
            pltpu.SemaphoreType.DMA((_NSLOT,)),
            pltpu.SemaphoreType.DMA((_NSLOT,)),
            pltpu.SemaphoreType.DMA((_NSLOT,)),
            pltpu.SemaphoreType.DMA((2,)),
            pltpu.SemaphoreType.DMA,
        ],
    )
    def sc_kernel(
        idx_hbm, pe_hbm, table_hbm, out_hbm,
        idx_v, rows_v, pe_sh, fsem, gsem, wsem, isem, sem0,
    ):
        wid = lax.axis_index("s") * nc + lax.axis_index("c")
        base = wid * n_chunks

        @pl.when(lax.axis_index("s") == 0)
        def _():
            pltpu.async_copy(pe_hbm, pe_sh, sem0).wait()

        plsc.subcore_barrier()

        def fire_idx(j, k):
            pltpu.make_async_copy(
                idx_hbm.at[pl.ds(base + j * _GRP, _GRP)], idx_v.at[k], isem.at[k]
            ).start()

        def wait_idx(k):
            pltpu.make_async_copy(
                idx_hbm.at[pl.ds(0, _GRP)], idx_v.at[k], isem.at[k]
            ).wait()

        def slot_ref(b):
            return rows_v.at[pl.ds(b * _SEQ, _SEQ)]

        def fire_fill(b):
            pltpu.make_async_copy(pe_sh, slot_ref(b), fsem.at[b]).start()

        def wait_fill(b):
            pltpu.make_async_copy(pe_sh, slot_ref(b), fsem.at[b]).wait()

        def fire_gather(k, b):
            pltpu.make_async_copy(
                table_hbm.at[idx_v.at[k, b, pl.ds(0, 128)]],
                rows_v.at[pl.ds(b * _SEQ, 128)],
                gsem.at[b],
            ).start(add=True)
            pltpu.make_async_copy(
                table_hbm.at[idx_v.at[k, b, pl.ds(128, _SEQ - 128)]],
                rows_v.at[pl.ds(b * _SEQ + 128, _SEQ - 128)],
                gsem.at[b],
            ).start(add=True)

        def wait_gather(b):
            pltpu.make_async_copy(
                table_hbm.at[pl.ds(0, 128)],
                rows_v.at[pl.ds(b * _SEQ, 128)],
                gsem.at[b],
            ).wait()
            pltpu.make_async_copy(
                table_hbm.at[pl.ds(0, _SEQ - 128)],
                rows_v.at[pl.ds(b * _SEQ + 128, _SEQ - 128)],
                gsem.at[b],
            ).wait()

        def fire_write(i, b):
            pltpu.make_async_copy(
                slot_ref(b),
                out_hbm.at[base + i, pl.ds(0, _SEQ), pl.ds(0, _DIM)],
                wsem.at[b],
            ).start()

        def wait_write(b):
            pltpu.make_async_copy(
                slot_ref(b),
                out_hbm.at[0, pl.ds(0, _SEQ), pl.ds(0, _DIM)],
                wsem.at[b],
            ).wait()

        fire_idx(0, 0)
        fire_idx(1, 1)
        fire_fill(0)
        fire_fill(1)

        def group_body(j, carry):
            k = lax.rem(j, 2)
            wait_idx(k)
            for b in range(_GRP):
                i = j * _GRP + b
                wait_fill(b)
                fire_gather(k, b)
                bp = (b - 1) % _NSLOT

                @pl.when(i > 0)
                def _():
                    wait_gather(bp)
                    fire_write(i - 1, bp)

                if b == 0:
                    @pl.when((j >= 1) & (j + 1 < n_groups))
                    def _():
                        fire_idx(j + 1, 1 - k)

                bn = (b + 2) % _NSLOT

                @pl.when(i + 2 < n_chunks)
                def _():
                    @pl.when(i >= 2)
                    def _():
                        wait_write(bn)

                    fire_fill(bn)

            return carry

        lax.fori_loop(0, n_groups, group_body, 0)

        blast = (n_chunks - 1) % _NSLOT
        wait_gather(blast)
        fire_write(n_chunks - 1, blast)
        for b in range(_NSLOT):
            wait_write(b)

    return sc_kernel


def _transpose_pad_table(table):
    v = table.shape[0]
    tt = table.T
    blk = 8192
    grid = (v + blk - 1) // blk

    def body(in_ref, out_ref):
        out_ref[:, 0:_DIM] = in_ref[...].T

    return pl.pallas_call(
        body,
        grid=(grid,),
        in_specs=[pl.BlockSpec((_DIM, blk), lambda i: (0, i))],
        out_specs=pl.BlockSpec((blk, _WIDE), lambda i: (i, 0)),
        out_shape=jax.ShapeDtypeStruct((v, _WIDE), jnp.float32),
    )(tt)


def kernel(input_ids, table):
    batch, seq = input_ids.shape
    ids = input_ids if input_ids.dtype == jnp.int32 else input_ids.astype(jnp.int32)
    pe = _positional_encoding(seq, table.shape[-1]).astype(jnp.float32)
    n_chunks = batch // 32
    tpad = _transpose_pad_table(table)
    trows = jnp.reshape(tpad, (2 * table.shape[0], _DIM))
    ids2 = ids * 2
    sc = _make_sc_kernel(batch, n_chunks)
    out = sc(ids2, pe, trows)
    return lax.slice(out, (0, 0, 0), (batch, seq, _DIM))


# speedup vs baseline: 3.0232x; 1.0320x over previous
import functools

import jax
import jax.numpy as jnp
from jax import lax
from jax.experimental import pallas as pl
from jax.experimental.pallas import tpu as pltpu
from jax.experimental.pallas import tpu_sc as plsc

_DIM = 64
_WIDE = 128
_SEQ = 200
_NSLOT = 4
_GRP = 4


def _positional_encoding(max_sequence_length, d_model):
    positions = jnp.arange(max_sequence_length)[:, None].astype(jnp.float32)
    dims = jnp.arange(d_model)[None, :]
    angle_rates = 1.0 / jnp.power(
        10000.0, (2 * (dims // 2)).astype(jnp.float32) / d_model
    )
    angle_rads = positions * angle_rates
    pe = jnp.zeros_like(angle_rads)
    pe = pe.at[:, 0::2].set(jnp.sin(angle_rads[:, 0::2]))
    pe = pe.at[:, 1::2].set(jnp.cos(angle_rads[:, 1::2]))
    return pe


def _make_sc_kernel(batch, n_chunks):
    info = plsc.get_sparse_core_info()
    nc, ns = info.num_cores, info.num_subcores
    mesh = plsc.VectorSubcoreMesh(core_axis_name="c", subcore_axis_name="s")
    n_groups = n_chunks // _GRP

    @functools.partial(
        pl.kernel,
        mesh=mesh,
        compiler_params=pltpu.CompilerParams(use_tc_tiling_on_sc=False),
        out_type=jax.ShapeDtypeStruct((batch, _SEQ, _WIDE), jnp.float32),
        scratch_types=[
            pltpu.VMEM((2, _GRP, _SEQ), jnp.int32),
            pltpu.VMEM((_NSLOT * _SEQ, _DIM), jnp.float32),
            pltpu.VMEM_SHARED((_SEQ, _DIM), jnp.float32),
            pltpu.SemaphoreType.DMA((_NSLOT,)),
            pltpu.SemaphoreType.DMA((_NSLOT,)),
            pltpu.SemaphoreType.DMA((_NSLOT,)),
            pltpu.SemaphoreType.DMA((2,)),
            pltpu.SemaphoreType.DMA,
        ],
    )
    def sc_kernel(
        idx_hbm, pe_hbm, table_hbm, out_hbm,
        idx_v, rows_v, pe_sh, fsem, gsem, wsem, isem, sem0,
    ):
        wid = lax.axis_index("s") * nc + lax.axis_index("c")
        base = wid * n_chunks

        @pl.when(lax.axis_index("s") == 0)
        def _():
            pltpu.async_copy(pe_hbm, pe_sh, sem0).wait()

        plsc.subcore_barrier()

        def fire_idx(j, k):
            pltpu.make_async_copy(
                idx_hbm.at[pl.ds(base + j * _GRP, _GRP)], idx_v.at[k], isem.at[k]
            ).start()

        def wait_idx(k):
            pltpu.make_async_copy(
                idx_hbm.at[pl.ds(0, _GRP)], idx_v.at[k], isem.at[k]
            ).wait()

        def slot_ref(b):
            return rows_v.at[pl.ds(b * _SEQ, _SEQ)]

        def fire_fill(b):
            pltpu.make_async_copy(pe_sh, slot_ref(b), fsem.at[b]).start()

        def wait_fill(b):
            pltpu.make_async_copy(pe_sh, slot_ref(b), fsem.at[b]).wait()

        def fire_gather(k, b):
            pltpu.make_async_copy(
                table_hbm.at[idx_v.at[k, b, pl.ds(0, 128)]],
                rows_v.at[pl.ds(b * _SEQ, 128)],
                gsem.at[b],
            ).start(add=True)
            pltpu.make_async_copy(
                table_hbm.at[idx_v.at[k, b, pl.ds(128, _SEQ - 128)]],
                rows_v.at[pl.ds(b * _SEQ + 128, _SEQ - 128)],
                gsem.at[b],
            ).start(add=True)

        def wait_gather(b):
            pltpu.make_async_copy(
                table_hbm.at[pl.ds(0, 128)],
                rows_v.at[pl.ds(b * _SEQ, 128)],
                gsem.at[b],
            ).wait()
            pltpu.make_async_copy(
                table_hbm.at[pl.ds(0, _SEQ - 128)],
                rows_v.at[pl.ds(b * _SEQ + 128, _SEQ - 128)],
                gsem.at[b],
            ).wait()

        def fire_write(i, b):
            pltpu.make_async_copy(
                slot_ref(b),
                out_hbm.at[base + i, pl.ds(0, _SEQ), pl.ds(0, _DIM)],
                wsem.at[b],
            ).start()

        def wait_write(b):
            pltpu.make_async_copy(
                slot_ref(b),
                out_hbm.at[0, pl.ds(0, _SEQ), pl.ds(0, _DIM)],
                wsem.at[b],
            ).wait()

        fire_idx(0, 0)
        fire_idx(1, 1)
        fire_fill(0)
        fire_fill(1)

        def group_body(j, carry):
            k = lax.rem(j, 2)
            wait_idx(k)
            for b in range(_GRP):
                i = j * _GRP + b
                wait_fill(b)
                fire_gather(k, b)
                bp = (b - 1) % _NSLOT

                @pl.when(i > 0)
                def _():
                    wait_gather(bp)
                    fire_write(i - 1, bp)

                if b == 0:
                    @pl.when((j >= 1) & (j + 1 < n_groups))
                    def _():
                        fire_idx(j + 1, 1 - k)

                bn = (b + 2) % _NSLOT

                @pl.when(i + 2 < n_chunks)
                def _():
                    @pl.when(i >= 2)
                    def _():
                        wait_write(bn)

                    fire_fill(bn)

            return carry

        lax.fori_loop(0, n_groups, group_body, 0)

        blast = (n_chunks - 1) % _NSLOT
        wait_gather(blast)
        fire_write(n_chunks - 1, blast)
        for b in range(_NSLOT):
            wait_write(b)

    return sc_kernel


def _transpose_pad_table(table):
    v = table.shape[0]
    tt = table.T
    blk = 16384
    grid = (v + blk - 1) // blk

    def body(in_ref, out_ref):
        out_ref[:, 0:_DIM] = in_ref[...].T

    return pl.pallas_call(
        body,
        grid=(grid,),
        in_specs=[pl.BlockSpec((_DIM, blk), lambda i: (0, i))],
        out_specs=pl.BlockSpec((blk, _WIDE), lambda i: (i, 0)),
        out_shape=jax.ShapeDtypeStruct((v, _WIDE), jnp.float32),
    )(tt)


def kernel(input_ids, table):
    batch, seq = input_ids.shape
    ids = input_ids if input_ids.dtype == jnp.int32 else input_ids.astype(jnp.int32)
    pe = _positional_encoding(seq, table.shape[-1]).astype(jnp.float32)
    n_chunks = batch // 32
    tpad = _transpose_pad_table(table)
    trows = jnp.reshape(tpad, (2 * table.shape[0], _DIM))
    ids2 = ids * 2
    sc = _make_sc_kernel(batch, n_chunks)
    out = sc(ids2, pe, trows)
    return lax.slice(out, (0, 0, 0), (batch, seq, _DIM))


# speedup vs baseline: 3.0633x; 1.0133x over previous
import functools

import jax
import jax.numpy as jnp
from jax import lax
from jax.experimental import pallas as pl
from jax.experimental.pallas import tpu as pltpu
from jax.experimental.pallas import tpu_sc as plsc

_DIM = 64
_WIDE = 128
_SEQ = 200
_NSLOT = 4
_GRP = 4


def _positional_encoding(max_sequence_length, d_model):
    positions = jnp.arange(max_sequence_length)[:, None].astype(jnp.float32)
    dims = jnp.arange(d_model)[None, :]
    angle_rates = 1.0 / jnp.power(
        10000.0, (2 * (dims // 2)).astype(jnp.float32) / d_model
    )
    angle_rads = positions * angle_rates
    pe = jnp.zeros_like(angle_rads)
    pe = pe.at[:, 0::2].set(jnp.sin(angle_rads[:, 0::2]))
    pe = pe.at[:, 1::2].set(jnp.cos(angle_rads[:, 1::2]))
    return pe


def _make_sc_kernel(batch, n_chunks):
    info = plsc.get_sparse_core_info()
    nc, ns = info.num_cores, info.num_subcores
    mesh = plsc.VectorSubcoreMesh(core_axis_name="c", subcore_axis_name="s")
    n_groups = n_chunks // _GRP

    @functools.partial(
        pl.kernel,
        mesh=mesh,
        compiler_params=pltpu.CompilerParams(use_tc_tiling_on_sc=False),
        out_type=jax.ShapeDtypeStruct((batch, _SEQ, _WIDE), jnp.float32),
        scratch_types=[
            pltpu.VMEM((2, _GRP, _SEQ), jnp.int32),
            pltpu.VMEM((_NSLOT * _SEQ, _DIM), jnp.float32),
            pltpu.VMEM_SHARED((_SEQ, _DIM), jnp.float32),
            pltpu.SemaphoreType.DMA((_NSLOT,)),
            pltpu.SemaphoreType.DMA((_NSLOT,)),
            pltpu.SemaphoreType.DMA((_NSLOT,)),
            pltpu.SemaphoreType.DMA((2,)),
            pltpu.SemaphoreType.DMA,
        ],
    )
    def sc_kernel(
        idx_hbm, pe_hbm, table_hbm, out_hbm,
        idx_v, rows_v, pe_sh, fsem, gsem, wsem, isem, sem0,
    ):
        wid = lax.axis_index("s") * nc + lax.axis_index("c")
        base = wid * n_chunks

        @pl.when(lax.axis_index("s") == 0)
        def _():
            pltpu.async_copy(pe_hbm, pe_sh, sem0).wait()

        plsc.subcore_barrier()

        def fire_idx(j, k):
            pltpu.make_async_copy(
                idx_hbm.at[pl.ds(base + j * _GRP, _GRP)], idx_v.at[k], isem.at[k]
            ).start()

        def wait_idx(k):
            pltpu.make_async_copy(
                idx_hbm.at[pl.ds(0, _GRP)], idx_v.at[k], isem.at[k]
            ).wait()

        def slot_ref(b):
            return rows_v.at[pl.ds(b * _SEQ, _SEQ)]

        def fire_fill(b):
            pltpu.make_async_copy(pe_sh, slot_ref(b), fsem.at[b]).start()

        def wait_fill(b):
            pltpu.make_async_copy(pe_sh, slot_ref(b), fsem.at[b]).wait()

        def fire_gather(k, b):
            pltpu.make_async_copy(
                table_hbm.at[idx_v.at[k, b, pl.ds(0, 128)]],
                rows_v.at[pl.ds(b * _SEQ, 128)],
                gsem.at[b],
            ).start(add=True)
            pltpu.make_async_copy(
                table_hbm.at[idx_v.at[k, b, pl.ds(128, _SEQ - 128)]],
                rows_v.at[pl.ds(b * _SEQ + 128, _SEQ - 128)],
                gsem.at[b],
            ).start(add=True)

        def wait_gather(b):
            pltpu.make_async_copy(
                table_hbm.at[pl.ds(0, 128)],
                rows_v.at[pl.ds(b * _SEQ, 128)],
                gsem.at[b],
            ).wait()
            pltpu.make_async_copy(
                table_hbm.at[pl.ds(0, _SEQ - 128)],
                rows_v.at[pl.ds(b * _SEQ + 128, _SEQ - 128)],
                gsem.at[b],
            ).wait()

        def fire_write(i, b):
            pltpu.make_async_copy(
                slot_ref(b),
                out_hbm.at[base + i, pl.ds(0, _SEQ), pl.ds(0, _DIM)],
                wsem.at[b],
            ).start()

        def wait_write(b):
            pltpu.make_async_copy(
                slot_ref(b),
                out_hbm.at[0, pl.ds(0, _SEQ), pl.ds(0, _DIM)],
                wsem.at[b],
            ).wait()

        fire_idx(0, 0)
        fire_idx(1, 1)
        fire_fill(0)
        fire_fill(1)

        def group_body(j, carry):
            k = lax.rem(j, 2)
            wait_idx(k)
            for b in range(_GRP):
                i = j * _GRP + b
                wait_fill(b)
                fire_gather(k, b)
                bp = (b - 1) % _NSLOT

                @pl.when(i > 0)
                def _():
                    wait_gather(bp)
                    fire_write(i - 1, bp)

                if b == 0:
                    @pl.when((j >= 1) & (j + 1 < n_groups))
                    def _():
                        fire_idx(j + 1, 1 - k)

                bn = (b + 2) % _NSLOT

                @pl.when(i + 2 < n_chunks)
                def _():
                    @pl.when(i >= 2)
                    def _():
                        wait_write(bn)

                    fire_fill(bn)

            return carry

        lax.fori_loop(0, n_groups, group_body, 0)

        blast = (n_chunks - 1) % _NSLOT
        wait_gather(blast)
        fire_write(n_chunks - 1, blast)
        for b in range(_NSLOT):
            wait_write(b)

    return sc_kernel


def _transpose_pad_table(table):
    v = table.shape[0]
    tt = table.T
    blk = 32768
    grid = (v + blk - 1) // blk

    def body(in_ref, out_ref):
        out_ref[:, 0:_DIM] = in_ref[...].T

    return pl.pallas_call(
        body,
        grid=(grid,),
        in_specs=[pl.BlockSpec((_DIM, blk), lambda i: (0, i))],
        out_specs=pl.BlockSpec((blk, _WIDE), lambda i: (i, 0)),
        out_shape=jax.ShapeDtypeStruct((v, _WIDE), jnp.float32),
    )(tt)


def kernel(input_ids, table):
    batch, seq = input_ids.shape
    ids = input_ids if input_ids.dtype == jnp.int32 else input_ids.astype(jnp.int32)
    pe = _positional_encoding(seq, table.shape[-1]).astype(jnp.float32)
    n_chunks = batch // 32
    tpad = _transpose_pad_table(table)
    trows = jnp.reshape(tpad, (2 * table.shape[0], _DIM))
    ids2 = ids * 2
    sc = _make_sc_kernel(batch, n_chunks)
    out = sc(ids2, pe, trows)
    return lax.slice(out, (0, 0, 0), (batch, seq, _DIM))
